# baseline (speedup 1.0000x reference)
import jax
import jax.numpy as jnp
from jax import lax
from jax.experimental import pallas as pl
from jax.experimental.pallas import tpu as pltpu
from jax.experimental.pallas import tpu_sc as plsc

_B = 4096
_D = 64
_NC = 2
_NS = 16
_NW = _NC * _NS
_BPW = _B // _NW
_NG = _BPW // 16

_LN2 = 0.6931471805599453
_LN100 = 4.605170185988092
_LOG2_POLY = (
    0.014778755424481588,
    -0.07684890405801897,
    0.1904211707113626,
    -0.32311624947178846,
    0.4724996763418957,
    -0.7203866484224759,
    1.4426521148584406,
    3.1958385927744075e-07,
)


def _ln(x):
    bits = plsc.bitcast(x, jnp.int32)
    e = ((bits >> 23) & 0xFF) - 127
    m = plsc.bitcast((bits & 0x007FFFFF) | 0x3F800000, jnp.float32)
    t = m - 1.0
    p = jnp.full((16,), _LOG2_POLY[0], dtype=jnp.float32)
    for coef in _LOG2_POLY[1:]:
        p = p * t + coef
    return (e.astype(jnp.float32) + p) * _LN2


def _sc_body(c_data, p_data, labels, c_table, c_bias, p_table, p_bias,
             out, cidx_v, pidx_v, lam_v, crows_v, prows_v, cb_v, pb_v,
             pvec_v, allv_v, outv_v, shared, sems):
    cid = lax.axis_index("c")
    sid = lax.axis_index("s")
    wid = cid * _NS + sid
    base = wid * _BPW

    pltpu.sync_copy(c_data.at[pl.ds(base, _BPW)], cidx_v)
    pltpu.sync_copy(p_data.at[pl.ds(base, _BPW)], pidx_v)
    cps = [
        pltpu.async_copy(c_table.at[cidx_v], crows_v, sems.at[0]),
        pltpu.async_copy(p_table.at[pidx_v], prows_v, sems.at[1]),
        pltpu.async_copy(c_bias.at[cidx_v], cb_v, sems.at[2]),
        pltpu.async_copy(p_bias.at[pidx_v], pb_v, sems.at[3]),
        pltpu.async_copy(labels.at[pl.ds(base, _BPW)], lam_v, sems.at[4]),
    ]
    for cp in cps:
        cp.wait()

    lane = lax.iota(jnp.int32, 16)

    acc_w = jnp.zeros((16,), jnp.float32)
    acc_a = jnp.zeros((16,), jnp.float32)
    acc_a2 = jnp.zeros((16,), jnp.float32)
    s_wb = jnp.float32(0.0)
    s_wb2 = jnp.float32(0.0)
    for g in range(_NG):
        lam = lam_v[pl.ds(g * 16, 16)]
        lnl = _ln(lam)
        w = jnp.minimum(jnp.exp(0.75 * (lnl - _LN100)), 1.0)
        acc_w += w
        a16 = cb_v[pl.ds(g * 16, 16)] + pb_v[pl.ds(g * 16, 16)]
        acc_a += a16
        acc_a2 += a16 * a16
        for jj in range(16):
            j = g * 16 + jj
            v = crows_v[j, pl.ds(0, 16)] * prows_v[j, pl.ds(0, 16)]
            for kk in range(1, _D // 16):
                v += (crows_v[j, pl.ds(kk * 16, 16)]
                      * prows_v[j, pl.ds(kk * 16, 16)])
            dot = jnp.sum(v)
            bv = dot - lnl[jj]
            wj = w[jj]
            s_wb += wj * bv
            s_wb2 += wj * (bv * bv)
    s_w = jnp.sum(acc_w)
    s_a = jnp.sum(acc_a)
    s_a2 = jnp.sum(acc_a2)

    packed = jnp.where(lane == 0, s_w, 0.0)
    packed = jnp.where(lane == 1, s_wb, packed)
    packed = jnp.where(lane == 2, s_wb2, packed)
    packed = jnp.where(lane == 3, s_a, packed)
    packed = jnp.where(lane == 4, s_a2, packed)
    pvec_v[...] = packed.astype(jnp.float32)

    pltpu.sync_copy(pvec_v, shared.at[sid])
    plsc.subcore_barrier()

    @pl.when(sid == 0)
    def _():
        pltpu.sync_copy(shared, allv_v)
        tot = allv_v[0, :]
        for k in range(1, _NS):
            tot += allv_v[k, :]
        outv_v[...] = tot
        pltpu.sync_copy(outv_v, out.at[cid])


@jax.jit
def kernel(c_data, p_data, labels, c_table, c_bias, p_table, p_bias):
    mesh = plsc.VectorSubcoreMesh(core_axis_name="c", subcore_axis_name="s")
    partials = pl.kernel(
        _sc_body,
        out_type=jax.ShapeDtypeStruct((_NC, 16), jnp.float32),
        mesh=mesh,
        compiler_params=pltpu.CompilerParams(
            needs_layout_passes=False, use_tc_tiling_on_sc=False),
        scratch_types=[
            pltpu.VMEM((_BPW,), jnp.int32),
            pltpu.VMEM((_BPW,), jnp.int32),
            pltpu.VMEM((_BPW,), jnp.float32),
            pltpu.VMEM((_BPW, _D), jnp.float32),
            pltpu.VMEM((_BPW, _D), jnp.float32),
            pltpu.VMEM((_BPW,), jnp.float32),
            pltpu.VMEM((_BPW,), jnp.float32),
            pltpu.VMEM((16,), jnp.float32),
            pltpu.VMEM((_NS, 16), jnp.float32),
            pltpu.VMEM((16,), jnp.float32),
            pltpu.VMEM_SHARED((_NS, 16), jnp.float32),
            pltpu.SemaphoreType.DMA((5,)),
        ],
    )(c_data, p_data, labels, c_table,
      jnp.reshape(c_bias, (-1,)), p_table, jnp.reshape(p_bias, (-1,)))
    tot = partials[0] + partials[1]
    bf = jnp.float32(_B)
    loss = (tot[0] * tot[4] + 2.0 * tot[3] * tot[1] + bf * tot[2]) / (bf * bf)
    return loss
